# baseline (speedup 1.0000x reference)
import functools

import jax
import jax.numpy as jnp
from jax import lax
from jax.experimental import pallas as pl
from jax.experimental.pallas import tpu as pltpu
from jax.experimental.pallas import tpu_sc as plsc

DIM = 1024
E = 8
CAP = 1280
N_TOK = 8192
EC = E * CAP

BLK = 1024
NB = N_TOK // BLK

NW = 32
TOK_W = N_TOK // NW
SLOT_W = EC // NW
SLOT_C = EC // 16
ROWS_C = (N_TOK // 128) // 16
GCHUNK = 32
CCHUNK = 16

_NEG_INF = float("-inf")


def _router_body(x_ref, wr_ref, br_ref,
                 pos1_ref, pos2_ref, sval1_ref, sval2_ref,
                 u1_ref, u2_ref, loss_ref, dist_ref, cnt_ref):
    b = pl.program_id(0)

    @pl.when(b == 0)
    def _init():
        cnt_ref[...] = jnp.zeros((E, 128), jnp.float32)

    xb = x_ref[...]
    wr = wr_ref[...]
    logits = lax.dot_general(wr, xb, (((1,), (1,)), ((), ())),
                             preferred_element_type=jnp.float32)
    logits = logits + br_ref[...][:, 0:1]

    eidx = lax.broadcasted_iota(jnp.int32, (E, BLK), 0)
    m1 = jnp.max(logits, axis=0, keepdims=True)
    a1 = jnp.min(jnp.where(logits == m1, eidx, E), axis=0, keepdims=True)
    h1 = eidx == a1
    masked = jnp.where(h1, _NEG_INF, logits)
    m2 = jnp.max(masked, axis=0, keepdims=True)
    a2 = jnp.min(jnp.where(masked == m2, eidx, E), axis=0, keepdims=True)
    h2 = eidx == a2

    t = jnp.exp(m2 - m1)
    den = 1.0 + t
    g1 = 1.0 / den
    g2 = t / den

    mask = h1.astype(jnp.float32) + h2.astype(jnp.float32)
    tri = (lax.broadcasted_iota(jnp.int32, (BLK, BLK), 0)
           < lax.broadcasted_iota(jnp.int32, (BLK, BLK), 1)).astype(jnp.float32)
    excl = lax.dot_general(mask, tri, (((1,), (0,)), ((), ())),
                           preferred_element_type=jnp.float32)
    base = cnt_ref[...][:, 0:1]
    s = excl + base
    cnt_new = base + jnp.sum(mask, axis=1, keepdims=True)
    cnt_ref[...] = jnp.broadcast_to(cnt_new, (E, 128))

    s1 = jnp.sum(jnp.where(h1, s, 0.0), axis=0, keepdims=True)
    s2 = jnp.sum(jnp.where(h2, s, 0.0), axis=0, keepdims=True)
    s1i = s1.astype(jnp.int32)
    s2i = s2.astype(jnp.int32)
    v1 = s1i < CAP
    v2 = s2i < CAP
    tok = b * BLK + lax.broadcasted_iota(jnp.int32, (1, BLK), 1)

    pos1_ref[...] = jnp.where(v1, a1 * CAP + s1i, 0).reshape(1, 1, BLK)
    pos2_ref[...] = jnp.where(v2, a2 * CAP + s2i, 0).reshape(1, 1, BLK)
    sval1_ref[...] = jnp.where(v1, tok, 0).reshape(1, 1, BLK)
    sval2_ref[...] = jnp.where(v2, tok, 0).reshape(1, 1, BLK)
    u1_ref[...] = jnp.where(v1, g1, 0.0).reshape(1, 1, BLK)
    u2_ref[...] = jnp.where(v2, g2, 0.0).reshape(1, 1, BLK)

    @pl.when(b == NB - 1)
    def _stats():
        cnt = cnt_ref[...]
        load = jnp.minimum(cnt, float(CAP))
        tot = jnp.sum(load[:, 0:1], axis=0, keepdims=True)
        dist = load / (tot + 1e-8)
        loss = -jnp.sum(dist[:, 0:1] * jnp.log(dist[:, 0:1] + 1e-8),
                        axis=0, keepdims=True)
        dist_ref[...] = dist
        loss_ref[...] = jnp.broadcast_to(loss, (8, 128))


def _run_router(x, wr, br_bc):
    outs = pl.pallas_call(
        _router_body,
        grid=(NB,),
        in_specs=[
            pl.BlockSpec((BLK, DIM), lambda b: (b, 0)),
            pl.BlockSpec((E, DIM), lambda b: (0, 0)),
            pl.BlockSpec((E, 128), lambda b: (0, 0)),
        ],
        out_specs=[
            pl.BlockSpec((1, 1, BLK), lambda b: (b, 0, 0)),
            pl.BlockSpec((1, 1, BLK), lambda b: (b, 0, 0)),
            pl.BlockSpec((1, 1, BLK), lambda b: (b, 0, 0)),
            pl.BlockSpec((1, 1, BLK), lambda b: (b, 0, 0)),
            pl.BlockSpec((1, 1, BLK), lambda b: (b, 0, 0)),
            pl.BlockSpec((1, 1, BLK), lambda b: (b, 0, 0)),
            pl.BlockSpec((8, 128), lambda b: (0, 0)),
            pl.BlockSpec((E, 128), lambda b: (0, 0)),
        ],
        out_shape=[
            jax.ShapeDtypeStruct((NB, 1, BLK), jnp.int32),
            jax.ShapeDtypeStruct((NB, 1, BLK), jnp.int32),
            jax.ShapeDtypeStruct((NB, 1, BLK), jnp.int32),
            jax.ShapeDtypeStruct((NB, 1, BLK), jnp.int32),
            jax.ShapeDtypeStruct((NB, 1, BLK), jnp.float32),
            jax.ShapeDtypeStruct((NB, 1, BLK), jnp.float32),
            jax.ShapeDtypeStruct((8, 128), jnp.float32),
            jax.ShapeDtypeStruct((E, 128), jnp.float32),
        ],
        scratch_shapes=[pltpu.VMEM((E, 128), jnp.float32)],
    )(x, wr, br_bc)
    return outs


def _dispatch_body(pos1_hbm, pos2_hbm, sval1_hbm, sval2_hbm, x_hbm,
                   xe_hbm,
                   perm_sh, zbuf, pbuf1, pbuf2, vbuf1, vbuf2, idxb,
                   rb0, rb1, gs0, gs1, os0, os1):
    wid = lax.axis_index("s") * 2 + lax.axis_index("c")
    sid = lax.axis_index("s")

    for j in range(SLOT_C // 16):
        zbuf[pl.ds(j * 16, 16)] = jnp.zeros((16,), jnp.int32)
    pltpu.sync_copy(zbuf, perm_sh.at[pl.ds(sid * SLOT_C, SLOT_C)])
    plsc.subcore_barrier()

    r0 = sid * ROWS_C
    pltpu.sync_copy(pos1_hbm.at[pl.ds(r0, ROWS_C), :], pbuf1)
    pltpu.sync_copy(pos2_hbm.at[pl.ds(r0, ROWS_C), :], pbuf2)
    pltpu.sync_copy(sval1_hbm.at[pl.ds(r0, ROWS_C), :], vbuf1)
    pltpu.sync_copy(sval2_hbm.at[pl.ds(r0, ROWS_C), :], vbuf2)
    for j in range(ROWS_C):
        pltpu.sync_copy(vbuf1.at[j], perm_sh.at[pbuf1.at[j]], add=True)
        pltpu.sync_copy(vbuf2.at[j], perm_sh.at[pbuf2.at[j]], add=True)
    plsc.subcore_barrier()

    pltpu.sync_copy(perm_sh.at[pl.ds(wid * SLOT_W, SLOT_W)], idxb)
    rbs = (rb0, rb1)
    gsems = (gs0, gs1)
    osems = (os0, os1)
    nch = SLOT_W // GCHUNK

    def issue(c):
        return pltpu.async_copy(
            x_hbm.at[idxb.at[pl.ds(c * GCHUNK, GCHUNK)]],
            rbs[c % 2], gsems[c % 2])

    d = issue(0)
    wo = [None, None]
    for c in range(nch):
        dn = None
        if c + 1 < nch:
            if wo[(c + 1) % 2] is not None:
                wo[(c + 1) % 2].wait()
                wo[(c + 1) % 2] = None
            dn = issue(c + 1)
        d.wait()
        wo[c % 2] = pltpu.async_copy(
            rbs[c % 2],
            xe_hbm.at[pl.ds(wid * SLOT_W + c * GCHUNK, GCHUNK), :],
            osems[c % 2])
        d = dn
    wo[0].wait()
    wo[1].wait()


def _run_dispatch(pos1_r, pos2_r, sval1_r, sval2_r, x):
    mesh = plsc.VectorSubcoreMesh(core_axis_name="c", subcore_axis_name="s", num_cores=2, num_subcores=16)
    k = functools.partial(
        pl.kernel,
        mesh=mesh,
        out_type=jax.ShapeDtypeStruct((EC, DIM), jnp.float32),
        scratch_types=[
            pltpu.VMEM_SHARED((EC,), jnp.int32),
            pltpu.VMEM((SLOT_C,), jnp.int32),
            pltpu.VMEM((ROWS_C, 128), jnp.int32),
            pltpu.VMEM((ROWS_C, 128), jnp.int32),
            pltpu.VMEM((ROWS_C, 128), jnp.int32),
            pltpu.VMEM((ROWS_C, 128), jnp.int32),
            pltpu.VMEM((SLOT_W,), jnp.int32),
            pltpu.VMEM((GCHUNK, DIM), jnp.float32),
            pltpu.VMEM((GCHUNK, DIM), jnp.float32),
            pltpu.SemaphoreType.DMA,
            pltpu.SemaphoreType.DMA,
            pltpu.SemaphoreType.DMA,
            pltpu.SemaphoreType.DMA,
        ],
    )(_dispatch_body)
    return k(pos1_r, pos2_r, sval1_r, sval2_r, x)


def _ffn_body(xe_ref, w1_ref, b1_ref, w2_ref, b2_ref, eo_ref):
    xb = xe_ref[...].astype(jnp.bfloat16)
    w1 = w1_ref[0].astype(jnp.bfloat16)
    h = lax.dot_general(xb, w1, (((1,), (1,)), ((), ())),
                        preferred_element_type=jnp.float32)
    h = jnp.maximum(h + b1_ref[0], 0.0).astype(jnp.bfloat16)
    w2 = w2_ref[0].astype(jnp.bfloat16)
    o = lax.dot_general(h, w2, (((1,), (1,)), ((), ())),
                        preferred_element_type=jnp.float32)
    eo_ref[...] = o + b2_ref[0]


TBLK = 256
TPE = CAP // TBLK


def _run_ffn(xe, w1, b1r, w2, b2r):
    return pl.pallas_call(
        _ffn_body,
        grid=(E, TPE),
        in_specs=[
            pl.BlockSpec((TBLK, DIM), lambda e, t: (e * TPE + t, 0)),
            pl.BlockSpec((1, DIM, DIM), lambda e, t: (e, 0, 0)),
            pl.BlockSpec((1, 1, DIM), lambda e, t: (e, 0, 0)),
            pl.BlockSpec((1, DIM, DIM), lambda e, t: (e, 0, 0)),
            pl.BlockSpec((1, 1, DIM), lambda e, t: (e, 0, 0)),
        ],
        out_specs=pl.BlockSpec((TBLK, DIM), lambda e, t: (e * TPE + t, 0)),
        out_shape=jax.ShapeDtypeStruct((EC, DIM), jnp.float32),
    )(xe, w1, b1r, w2, b2r)


def _combine_body(eo_hbm, pos1_hbm, pos2_hbm, u1_hbm, u2_hbm,
                  out_hbm,
                  pbuf1, pbuf2, ubuf1, ubuf2, pcat,
                  rca, rcb, oba, obb,
                  sga, sgb, soa, sob):
    wid = lax.axis_index("s") * 2 + lax.axis_index("c")
    base = wid * TOK_W
    pltpu.sync_copy(pos1_hbm.at[pl.ds(base, TOK_W)], pbuf1)
    pltpu.sync_copy(pos2_hbm.at[pl.ds(base, TOK_W)], pbuf2)
    pltpu.sync_copy(u1_hbm.at[pl.ds(base, TOK_W)], ubuf1)
    pltpu.sync_copy(u2_hbm.at[pl.ds(base, TOK_W)], ubuf2)

    nch = TOK_W // CCHUNK
    for c in range(nch):
        pcat[pl.ds(2 * CCHUNK * c, CCHUNK)] = pbuf1[pl.ds(CCHUNK * c, CCHUNK)]
        pcat[pl.ds(2 * CCHUNK * c + CCHUNK, CCHUNK)] = (
            pbuf2[pl.ds(CCHUNK * c, CCHUNK)])

    def issue(c, rc, sg):
        pltpu.async_copy(
            eo_hbm.at[pcat.at[pl.ds(2 * CCHUNK * c, 2 * CCHUNK)]], rc, sg)

    def wait_gather(rc, sg):
        pltpu.make_async_copy(
            eo_hbm.at[pl.ds(0, 2 * CCHUNK), :], rc, sg).wait()

    def compute(c, rc, ob, so, first_pair):
        uv1 = ubuf1[pl.ds(c * CCHUNK, CCHUNK)]
        uv2 = ubuf2[pl.ds(c * CCHUNK, CCHUNK)]
        @pl.when(jnp.logical_not(first_pair))
        def _():
            pltpu.make_async_copy(
                ob, out_hbm.at[pl.ds(base, CCHUNK), :], so).wait()
        for t in range(CCHUNK):
            a = uv1[t]
            bb = uv2[t]

            def col8(j8, _):
                o = j8 * 128
                for k in range(8):
                    sl = pl.ds(o + k * 16, 16)
                    ob[t, sl] = a * rc[t, sl] + bb * rc[CCHUNK + t, sl]
                return 0

            lax.fori_loop(0, DIM // 128, col8, 0)
        pltpu.async_copy(ob, out_hbm.at[pl.ds(base + c * CCHUNK, CCHUNK), :],
                         so)

    issue(0, rca, sga)

    def pair(c8, _):
        c0 = 2 * c8
        first = c8 == 0
        issue(c0 + 1, rcb, sgb)
        wait_gather(rca, sga)
        compute(c0, rca, oba, soa, first)
        cn = jnp.minimum(c0 + 2, nch - 1)
        issue(cn, rca, sga)
        wait_gather(rcb, sgb)
        compute(c0 + 1, rcb, obb, sob, first)
        return 0

    lax.fori_loop(0, nch // 2, pair, 0)
    wait_gather(rca, sga)
    pltpu.make_async_copy(oba, out_hbm.at[pl.ds(base, CCHUNK), :], soa).wait()
    pltpu.make_async_copy(obb, out_hbm.at[pl.ds(base, CCHUNK), :], sob).wait()


def _run_combine(eo, pos1_f, pos2_f, u1_f, u2_f):
    mesh = plsc.VectorSubcoreMesh(core_axis_name="c", subcore_axis_name="s", num_cores=2, num_subcores=16)
    k = functools.partial(
        pl.kernel,
        mesh=mesh,
        out_type=jax.ShapeDtypeStruct((N_TOK, DIM), jnp.float32),
        scratch_types=[
            pltpu.VMEM((TOK_W,), jnp.int32),
            pltpu.VMEM((TOK_W,), jnp.int32),
            pltpu.VMEM((TOK_W,), jnp.float32),
            pltpu.VMEM((TOK_W,), jnp.float32),
            pltpu.VMEM((2 * TOK_W,), jnp.int32),
            pltpu.VMEM((2 * CCHUNK, DIM), jnp.float32),
            pltpu.VMEM((2 * CCHUNK, DIM), jnp.float32),
            pltpu.VMEM((CCHUNK, DIM), jnp.float32),
            pltpu.VMEM((CCHUNK, DIM), jnp.float32),
            pltpu.SemaphoreType.DMA,
            pltpu.SemaphoreType.DMA,
            pltpu.SemaphoreType.DMA,
            pltpu.SemaphoreType.DMA,
        ],
    )(_combine_body)
    return k(eo, pos1_f, pos2_f, u1_f, u2_f)


def kernel(x, Wr, br, W1, b1, W2, b2):
    br_bc = jnp.broadcast_to(br.reshape(E, 1), (E, 128))
    (pos1, pos2, sval1, sval2, u1, u2, loss_b, dist_b) = _run_router(
        x, Wr, br_bc)

    pos1_r = pos1.reshape(N_TOK // 128, 128)
    pos2_r = pos2.reshape(N_TOK // 128, 128)
    sval1_r = sval1.reshape(N_TOK // 128, 128)
    sval2_r = sval2.reshape(N_TOK // 128, 128)
    xe = _run_dispatch(pos1_r, pos2_r, sval1_r, sval2_r, x)

    eo = _run_ffn(xe, W1, b1.reshape(E, 1, DIM), W2, b2.reshape(E, 1, DIM))

    out = _run_combine(eo, pos1.reshape(N_TOK), pos2.reshape(N_TOK),
                       u1.reshape(N_TOK), u2.reshape(N_TOK))

    load_loss = loss_b[0, 0]
    load_dist = dist_b[:, 0]
    return out, load_loss, load_dist


# speedup vs baseline: 1.8985x; 1.8985x over previous
import functools

import jax
import jax.numpy as jnp
from jax import lax
from jax.experimental import pallas as pl
from jax.experimental.pallas import tpu as pltpu
from jax.experimental.pallas import tpu_sc as plsc

DIM = 1024
E = 8
CAP = 1280
N_TOK = 8192
EC = E * CAP

BLK = 1024
NB = N_TOK // BLK

NW = 32
TOK_W = N_TOK // NW
SLOT_W = EC // NW
SLOT_C = EC // 16
ROWS_C = (N_TOK // 128) // 16
GCHUNK = 32
CCHUNK = 16

_NEG_INF = float("-inf")


def _router_body(x_ref, wr_ref, br_ref,
                 pos1_ref, pos2_ref, sval1_ref, sval2_ref,
                 u1_ref, u2_ref, loss_ref, dist_ref, cnt_ref):
    b = pl.program_id(0)

    @pl.when(b == 0)
    def _init():
        cnt_ref[...] = jnp.zeros((E, 128), jnp.float32)

    xb = x_ref[...]
    wr = wr_ref[...]
    logits = lax.dot_general(wr, xb, (((1,), (1,)), ((), ())),
                             preferred_element_type=jnp.float32)
    logits = logits + br_ref[...][:, 0:1]

    eidx = lax.broadcasted_iota(jnp.int32, (E, BLK), 0)
    m1 = jnp.max(logits, axis=0, keepdims=True)
    a1 = jnp.min(jnp.where(logits == m1, eidx, E), axis=0, keepdims=True)
    h1 = eidx == a1
    masked = jnp.where(h1, _NEG_INF, logits)
    m2 = jnp.max(masked, axis=0, keepdims=True)
    a2 = jnp.min(jnp.where(masked == m2, eidx, E), axis=0, keepdims=True)
    h2 = eidx == a2

    t = jnp.exp(m2 - m1)
    den = 1.0 + t
    g1 = 1.0 / den
    g2 = t / den

    mask = h1.astype(jnp.float32) + h2.astype(jnp.float32)
    tri = (lax.broadcasted_iota(jnp.int32, (BLK, BLK), 0)
           < lax.broadcasted_iota(jnp.int32, (BLK, BLK), 1)).astype(jnp.float32)
    excl = lax.dot_general(mask, tri, (((1,), (0,)), ((), ())),
                           preferred_element_type=jnp.float32)
    base = cnt_ref[...][:, 0:1]
    s = excl + base
    cnt_new = base + jnp.sum(mask, axis=1, keepdims=True)
    cnt_ref[...] = jnp.broadcast_to(cnt_new, (E, 128))

    s1 = jnp.sum(jnp.where(h1, s, 0.0), axis=0, keepdims=True)
    s2 = jnp.sum(jnp.where(h2, s, 0.0), axis=0, keepdims=True)
    s1i = s1.astype(jnp.int32)
    s2i = s2.astype(jnp.int32)
    v1 = s1i < CAP
    v2 = s2i < CAP
    tok = b * BLK + lax.broadcasted_iota(jnp.int32, (1, BLK), 1)

    pos1_ref[...] = jnp.where(v1, a1 * CAP + s1i, 0).reshape(1, 1, BLK)
    pos2_ref[...] = jnp.where(v2, a2 * CAP + s2i, 0).reshape(1, 1, BLK)
    sval1_ref[...] = jnp.where(v1, tok, 0).reshape(1, 1, BLK)
    sval2_ref[...] = jnp.where(v2, tok, 0).reshape(1, 1, BLK)
    u1_ref[...] = jnp.where(v1, g1, 0.0).reshape(1, 1, BLK)
    u2_ref[...] = jnp.where(v2, g2, 0.0).reshape(1, 1, BLK)

    @pl.when(b == NB - 1)
    def _stats():
        cnt = cnt_ref[...]
        load = jnp.minimum(cnt, float(CAP))
        tot = jnp.sum(load[:, 0:1], axis=0, keepdims=True)
        dist = load / (tot + 1e-8)
        loss = -jnp.sum(dist[:, 0:1] * jnp.log(dist[:, 0:1] + 1e-8),
                        axis=0, keepdims=True)
        dist_ref[...] = dist
        loss_ref[...] = jnp.broadcast_to(loss, (8, 128))


def _run_router(x, wr, br_bc):
    outs = pl.pallas_call(
        _router_body,
        grid=(NB,),
        in_specs=[
            pl.BlockSpec((BLK, DIM), lambda b: (b, 0)),
            pl.BlockSpec((E, DIM), lambda b: (0, 0)),
            pl.BlockSpec((E, 128), lambda b: (0, 0)),
        ],
        out_specs=[
            pl.BlockSpec((1, 1, BLK), lambda b: (b, 0, 0)),
            pl.BlockSpec((1, 1, BLK), lambda b: (b, 0, 0)),
            pl.BlockSpec((1, 1, BLK), lambda b: (b, 0, 0)),
            pl.BlockSpec((1, 1, BLK), lambda b: (b, 0, 0)),
            pl.BlockSpec((1, 1, BLK), lambda b: (b, 0, 0)),
            pl.BlockSpec((1, 1, BLK), lambda b: (b, 0, 0)),
            pl.BlockSpec((8, 128), lambda b: (0, 0)),
            pl.BlockSpec((E, 128), lambda b: (0, 0)),
        ],
        out_shape=[
            jax.ShapeDtypeStruct((NB, 1, BLK), jnp.int32),
            jax.ShapeDtypeStruct((NB, 1, BLK), jnp.int32),
            jax.ShapeDtypeStruct((NB, 1, BLK), jnp.int32),
            jax.ShapeDtypeStruct((NB, 1, BLK), jnp.int32),
            jax.ShapeDtypeStruct((NB, 1, BLK), jnp.float32),
            jax.ShapeDtypeStruct((NB, 1, BLK), jnp.float32),
            jax.ShapeDtypeStruct((8, 128), jnp.float32),
            jax.ShapeDtypeStruct((E, 128), jnp.float32),
        ],
        scratch_shapes=[pltpu.VMEM((E, 128), jnp.float32)],
    )(x, wr, br_bc)
    return outs


def _dispatch_body(pos1_hbm, pos2_hbm, sval1_hbm, sval2_hbm,
                   u1_hbm, u2_hbm, x_hbm,
                   xe_hbm, p1o_hbm, p2o_hbm, w1o_hbm, w2o_hbm,
                   p1_sh, p2_sh, w1_sh, w2_sh, zbi, zbf,
                   pbuf1, pbuf2, vbuf1, vbuf2, ubuf1, ubuf2,
                   idxb, idxb2, wtmp1, wtmp2,
                   rb0, rb1, gs0, gs1, os0, os1):
    wid = lax.axis_index("s") * 2 + lax.axis_index("c")
    sid = lax.axis_index("s")

    for j in range(SLOT_C // 16):
        zbi[pl.ds(j * 16, 16)] = jnp.zeros((16,), jnp.int32)
        zbf[pl.ds(j * 16, 16)] = jnp.zeros((16,), jnp.float32)
    pltpu.sync_copy(zbi, p1_sh.at[pl.ds(sid * SLOT_C, SLOT_C)])
    pltpu.sync_copy(zbi, p2_sh.at[pl.ds(sid * SLOT_C, SLOT_C)])
    pltpu.sync_copy(zbf, w1_sh.at[pl.ds(sid * SLOT_C, SLOT_C)])
    pltpu.sync_copy(zbf, w2_sh.at[pl.ds(sid * SLOT_C, SLOT_C)])
    plsc.subcore_barrier()

    r0 = sid * ROWS_C
    pltpu.sync_copy(pos1_hbm.at[pl.ds(r0, ROWS_C), :], pbuf1)
    pltpu.sync_copy(pos2_hbm.at[pl.ds(r0, ROWS_C), :], pbuf2)
    pltpu.sync_copy(sval1_hbm.at[pl.ds(r0, ROWS_C), :], vbuf1)
    pltpu.sync_copy(sval2_hbm.at[pl.ds(r0, ROWS_C), :], vbuf2)
    pltpu.sync_copy(u1_hbm.at[pl.ds(r0, ROWS_C), :], ubuf1)
    pltpu.sync_copy(u2_hbm.at[pl.ds(r0, ROWS_C), :], ubuf2)
    for j in range(ROWS_C):
        pltpu.sync_copy(vbuf1.at[j], p1_sh.at[pbuf1.at[j]], add=True)
        pltpu.sync_copy(vbuf2.at[j], p2_sh.at[pbuf2.at[j]], add=True)
        pltpu.sync_copy(ubuf1.at[j], w1_sh.at[pbuf1.at[j]], add=True)
        pltpu.sync_copy(ubuf2.at[j], w2_sh.at[pbuf2.at[j]], add=True)
    plsc.subcore_barrier()

    sl_w = pl.ds(wid * SLOT_W, SLOT_W)
    pltpu.sync_copy(p1_sh.at[sl_w], idxb)
    pltpu.sync_copy(idxb, p1o_hbm.at[sl_w])
    pltpu.sync_copy(p2_sh.at[sl_w], idxb2)
    pltpu.sync_copy(idxb2, p2o_hbm.at[sl_w])
    pltpu.sync_copy(w1_sh.at[sl_w], wtmp1)
    pltpu.sync_copy(wtmp1, w1o_hbm.at[sl_w])
    pltpu.sync_copy(w2_sh.at[sl_w], wtmp2)
    pltpu.sync_copy(wtmp2, w2o_hbm.at[sl_w])

    for g in range(SLOT_W // 16):
        sl = pl.ds(g * 16, 16)
        idxb[sl] = idxb[sl] + idxb2[sl]
    rbs = (rb0, rb1)
    gsems = (gs0, gs1)
    osems = (os0, os1)
    nch = SLOT_W // GCHUNK

    def issue(c):
        return pltpu.async_copy(
            x_hbm.at[idxb.at[pl.ds(c * GCHUNK, GCHUNK)]],
            rbs[c % 2], gsems[c % 2])

    d = issue(0)
    wo = [None, None]
    for c in range(nch):
        dn = None
        if c + 1 < nch:
            if wo[(c + 1) % 2] is not None:
                wo[(c + 1) % 2].wait()
                wo[(c + 1) % 2] = None
            dn = issue(c + 1)
        d.wait()
        wo[c % 2] = pltpu.async_copy(
            rbs[c % 2],
            xe_hbm.at[pl.ds(wid * SLOT_W + c * GCHUNK, GCHUNK), :],
            osems[c % 2])
        d = dn
    wo[0].wait()
    wo[1].wait()


def _run_dispatch(pos1_r, pos2_r, sval1_r, sval2_r, u1_r, u2_r, x):
    mesh = plsc.VectorSubcoreMesh(core_axis_name="c", subcore_axis_name="s", num_cores=2, num_subcores=16)
    k = functools.partial(
        pl.kernel,
        mesh=mesh,
        out_type=[
            jax.ShapeDtypeStruct((EC, DIM), jnp.float32),
            jax.ShapeDtypeStruct((EC,), jnp.int32),
            jax.ShapeDtypeStruct((EC,), jnp.int32),
            jax.ShapeDtypeStruct((EC,), jnp.float32),
            jax.ShapeDtypeStruct((EC,), jnp.float32),
        ],
        scratch_types=[
            pltpu.VMEM_SHARED((EC,), jnp.int32),
            pltpu.VMEM_SHARED((EC,), jnp.int32),
            pltpu.VMEM_SHARED((EC,), jnp.float32),
            pltpu.VMEM_SHARED((EC,), jnp.float32),
            pltpu.VMEM((SLOT_C,), jnp.int32),
            pltpu.VMEM((SLOT_C,), jnp.float32),
            pltpu.VMEM((ROWS_C, 128), jnp.int32),
            pltpu.VMEM((ROWS_C, 128), jnp.int32),
            pltpu.VMEM((ROWS_C, 128), jnp.int32),
            pltpu.VMEM((ROWS_C, 128), jnp.int32),
            pltpu.VMEM((ROWS_C, 128), jnp.float32),
            pltpu.VMEM((ROWS_C, 128), jnp.float32),
            pltpu.VMEM((SLOT_W,), jnp.int32),
            pltpu.VMEM((SLOT_W,), jnp.int32),
            pltpu.VMEM((SLOT_W,), jnp.float32),
            pltpu.VMEM((SLOT_W,), jnp.float32),
            pltpu.VMEM((GCHUNK, DIM), jnp.float32),
            pltpu.VMEM((GCHUNK, DIM), jnp.float32),
            pltpu.SemaphoreType.DMA,
            pltpu.SemaphoreType.DMA,
            pltpu.SemaphoreType.DMA,
            pltpu.SemaphoreType.DMA,
        ],
    )(_dispatch_body)
    return k(pos1_r, pos2_r, sval1_r, sval2_r, u1_r, u2_r, x)


def _ffn_body(xe_ref, w1_ref, b1_ref, w2_ref, b2_ref, eo_ref):
    xb = xe_ref[...].astype(jnp.bfloat16)
    w1 = w1_ref[0].astype(jnp.bfloat16)
    h = lax.dot_general(xb, w1, (((1,), (1,)), ((), ())),
                        preferred_element_type=jnp.float32)
    h = jnp.maximum(h + b1_ref[0], 0.0).astype(jnp.bfloat16)
    w2 = w2_ref[0].astype(jnp.bfloat16)
    o = lax.dot_general(h, w2, (((1,), (1,)), ((), ())),
                        preferred_element_type=jnp.float32)
    eo_ref[...] = o + b2_ref[0]


TBLK = 256
TPE = CAP // TBLK


def _run_ffn(xe, w1, b1r, w2, b2r):
    return pl.pallas_call(
        _ffn_body,
        grid=(E, TPE),
        in_specs=[
            pl.BlockSpec((TBLK, DIM), lambda e, t: (e * TPE + t, 0)),
            pl.BlockSpec((1, DIM, DIM), lambda e, t: (e, 0, 0)),
            pl.BlockSpec((1, 1, DIM), lambda e, t: (e, 0, 0)),
            pl.BlockSpec((1, DIM, DIM), lambda e, t: (e, 0, 0)),
            pl.BlockSpec((1, 1, DIM), lambda e, t: (e, 0, 0)),
        ],
        out_specs=pl.BlockSpec((TBLK, DIM), lambda e, t: (e * TPE + t, 0)),
        out_shape=jax.ShapeDtypeStruct((EC, DIM), jnp.float32),
    )(xe, w1, b1r, w2, b2r)


OEXT_ROWS = 2 * N_TOK + 256
SCHUNK = 16
NCH_W = SLOT_W // SCHUNK


def _combine_body(eo_hbm, p1_hbm, p2_hbm, w1_hbm, w2_hbm,
                  oext_hbm,
                  p1b, p2b, w1b, w2b, wbuf, tbuf,
                  rba, rbb, oba, obb, sga, sgb, soa, sob):
    wid = lax.axis_index("s") * 2 + lax.axis_index("c")
    sbase = wid * SLOT_W
    pltpu.sync_copy(p1_hbm.at[pl.ds(sbase, SLOT_W)], p1b)
    pltpu.sync_copy(p2_hbm.at[pl.ds(sbase, SLOT_W)], p2b)
    pltpu.sync_copy(w1_hbm.at[pl.ds(sbase, SLOT_W)], w1b)
    pltpu.sync_copy(w2_hbm.at[pl.ds(sbase, SLOT_W)], w2b)

    for g in range(SLOT_W // 16):
        sl = pl.ds(g * 16, 16)
        w1v = w1b[sl]
        w2v = w2b[sl]
        wbuf[sl] = w1v + w2v
        tgt = jnp.where(w1v > 0.0, p1b[sl],
                        jnp.where(w2v > 0.0, p2b[sl] + N_TOK, 2 * N_TOK))
        tbuf[g, pl.ds(0, 16)] = tgt

    def issue(c, rb, sg):
        pltpu.async_copy(eo_hbm.at[pl.ds(sbase + c * SCHUNK, SCHUNK), :],
                         rb, sg)

    def wait_gather(rb, sg):
        pltpu.make_async_copy(eo_hbm.at[pl.ds(0, SCHUNK), :], rb, sg).wait()

    def compute(c, rb, ob, so, first_pair):
        wv = wbuf[pl.ds(c * SCHUNK, SCHUNK)]
        @pl.when(jnp.logical_not(first_pair))
        def _():
            pltpu.make_async_copy(
                ob, oext_hbm.at[pl.ds(0, SCHUNK), :], so).wait()
        for t in range(SCHUNK):
            a = wv[t]

            def col8(j8, _):
                o = j8 * 128
                for k in range(8):
                    sl = pl.ds(o + k * 16, 16)
                    ob[t, sl] = a * rb[t, sl]
                return 0

            lax.fori_loop(0, DIM // 128, col8, 0)
        pltpu.async_copy(ob, oext_hbm.at[tbuf.at[c]], so)

    issue(0, rba, sga)

    def pair(c8, _):
        c0 = 2 * c8
        first = c8 == 0
        issue(c0 + 1, rbb, sgb)
        wait_gather(rba, sga)
        compute(c0, rba, oba, soa, first)
        cn = jnp.minimum(c0 + 2, NCH_W - 1)
        issue(cn, rba, sga)
        wait_gather(rbb, sgb)
        compute(c0 + 1, rbb, obb, sob, first)
        return 0

    lax.fori_loop(0, NCH_W // 2, pair, 0)
    wait_gather(rba, sga)
    pltpu.make_async_copy(oba, oext_hbm.at[pl.ds(0, SCHUNK), :], soa).wait()
    pltpu.make_async_copy(obb, oext_hbm.at[pl.ds(0, SCHUNK), :], sob).wait()


def _run_combine(eo, p1v, p2v, w1v, w2v):
    mesh = plsc.VectorSubcoreMesh(core_axis_name="c", subcore_axis_name="s", num_cores=2, num_subcores=16)
    k = functools.partial(
        pl.kernel,
        mesh=mesh,
        out_type=jax.ShapeDtypeStruct((OEXT_ROWS, DIM), jnp.float32),
        scratch_types=[
            pltpu.VMEM((SLOT_W,), jnp.int32),
            pltpu.VMEM((SLOT_W,), jnp.int32),
            pltpu.VMEM((SLOT_W,), jnp.float32),
            pltpu.VMEM((SLOT_W,), jnp.float32),
            pltpu.VMEM((SLOT_W,), jnp.float32),
            pltpu.VMEM((NCH_W, SCHUNK), jnp.int32),
            pltpu.VMEM((SCHUNK, DIM), jnp.float32),
            pltpu.VMEM((SCHUNK, DIM), jnp.float32),
            pltpu.VMEM((SCHUNK, DIM), jnp.float32),
            pltpu.VMEM((SCHUNK, DIM), jnp.float32),
            pltpu.SemaphoreType.DMA,
            pltpu.SemaphoreType.DMA,
            pltpu.SemaphoreType.DMA,
            pltpu.SemaphoreType.DMA,
        ],
    )(_combine_body)
    return k(eo, p1v, p2v, w1v, w2v)


ABLK = 256


def _add_body(a_ref, b_ref, u1_ref, u2_ref, out_ref):
    a = a_ref[...]
    b = b_ref[...]
    m1 = u1_ref[...] > 0.0
    m2 = u2_ref[...] > 0.0
    out_ref[...] = (jnp.where(m1, a, 0.0) + jnp.where(m2, b, 0.0))


def _run_add(oext, u1_c, u2_c):
    nblk = N_TOK // ABLK
    return pl.pallas_call(
        _add_body,
        grid=(nblk,),
        in_specs=[
            pl.BlockSpec((ABLK, DIM), lambda i: (i, 0)),
            pl.BlockSpec((ABLK, DIM), lambda i: (i + N_TOK // ABLK, 0)),
            pl.BlockSpec((ABLK, 1), lambda i: (i, 0)),
            pl.BlockSpec((ABLK, 1), lambda i: (i, 0)),
        ],
        out_specs=pl.BlockSpec((ABLK, DIM), lambda i: (i, 0)),
        out_shape=jax.ShapeDtypeStruct((N_TOK, DIM), jnp.float32),
    )(oext, oext, u1_c, u2_c)


def kernel(x, Wr, br, W1, b1, W2, b2):
    br_bc = jnp.broadcast_to(br.reshape(E, 1), (E, 128))
    (pos1, pos2, sval1, sval2, u1, u2, loss_b, dist_b) = _run_router(
        x, Wr, br_bc)

    pos1_r = pos1.reshape(N_TOK // 128, 128)
    pos2_r = pos2.reshape(N_TOK // 128, 128)
    sval1_r = sval1.reshape(N_TOK // 128, 128)
    sval2_r = sval2.reshape(N_TOK // 128, 128)
    u1_r = u1.reshape(N_TOK // 128, 128)
    u2_r = u2.reshape(N_TOK // 128, 128)
    xe, p1v, p2v, w1v, w2v = _run_dispatch(
        pos1_r, pos2_r, sval1_r, sval2_r, u1_r, u2_r, x)

    eo = _run_ffn(xe, W1, b1.reshape(E, 1, DIM), W2, b2.reshape(E, 1, DIM))

    oext = _run_combine(eo, p1v, p2v, w1v, w2v)
    out = _run_add(oext, u1.reshape(N_TOK, 1), u2.reshape(N_TOK, 1))

    load_loss = loss_b[0, 0]
    load_dist = dist_b[:, 0]
    return out, load_loss, load_dist
